# baseline (speedup 1.0000x reference)
import functools

import jax
import jax.numpy as jnp
from jax import lax
from jax.experimental import pallas as pl
from jax.experimental.pallas import tpu as pltpu
from jax.experimental.pallas import tpu_sc as plsc

N_NODES = 100000
F = 128
E = 64
B = 4096
DEG = 32
K = 16
L = 16
NC, NS = 2, 16
NW = NC * NS
RPT = B // NW

_SCORE_BLK = 4096
_N_PAD = 102400


def _score_body(ft_ref, w_ref, out_ref):
    res = lax.dot_general(w_ref[...], ft_ref[...], (((0,), (1,)), ((), ())),
                          preferred_element_type=jnp.float32)
    out_ref[...] = res[0]


_score_scan = pl.pallas_call(
    _score_body,
    grid=(_N_PAD // _SCORE_BLK,),
    in_specs=[
        pl.BlockSpec((_SCORE_BLK, F), lambda i: (i, 0)),
        pl.BlockSpec((F, 1), lambda i: (0, 0)),
    ],
    out_specs=pl.BlockSpec((_SCORE_BLK,), lambda i: (i,)),
    out_shape=jax.ShapeDtypeStruct((_N_PAD,), jnp.float32),
)

_sc_mesh = plsc.VectorSubcoreMesh(core_axis_name="c", subcore_axis_name="s")


@functools.partial(
    pl.kernel,
    out_type=[
        jax.ShapeDtypeStruct((B, F), jnp.float32),
        jax.ShapeDtypeStruct((B, F), jnp.float32),
        jax.ShapeDtypeStruct((B, F), jnp.float32),
        jax.ShapeDtypeStruct((B, F), jnp.float32),
    ],
    mesh=_sc_mesh,
    compiler_params=pltpu.CompilerParams(needs_layout_passes=False),
    scratch_types=[
        pltpu.VMEM((RPT,), jnp.int32),
        pltpu.VMEM((RPT,), jnp.float32),
        pltpu.VMEM((RPT * DEG,), jnp.int32),
        pltpu.VMEM((RPT * DEG,), jnp.float32),
        pltpu.VMEM((RPT * K,), jnp.int32),
        pltpu.VMEM((128, F), jnp.float32),
        pltpu.VMEM((128, F), jnp.float32),
        pltpu.VMEM((128, F), jnp.float32),
        pltpu.VMEM((128, F), jnp.float32),
        pltpu.VMEM((64, F), jnp.float32),
        pltpu.VMEM_SHARED((_N_PAD,), jnp.float32),
        pltpu.SemaphoreType.DMA,
        pltpu.SemaphoreType.DMA,
        pltpu.SemaphoreType.DMA,
        pltpu.SemaphoreType.DMA,
        pltpu.SemaphoreType.DMA,
    ],
)
def _sc_select_agg(scores_hbm, nodes_hbm, n1_hbm, n2_hbm, n3_hbm, feat_hbm,
                   self_hbm, a1_hbm, a2_hbm, a3_hbm,
                   nodes_v, cent_v, neigh_v, nsc_v, sel_v,
                   rb0, rb1, rb2, rb3, agg_v, score_s,
                   sem0, sem1, sem2, sem3, sems5):
    sid = lax.axis_index("s")
    wid = sid * NC + lax.axis_index("c")
    base = wid * RPT

    rbufs = (rb0, rb1, rb2, rb3)
    sems = (sem0, sem1, sem2, sem3)

    _CH = _N_PAD // NS
    pltpu.sync_copy(scores_hbm.at[pl.ds(sid * _CH, _CH)],
                    score_s.at[pl.ds(sid * _CH, _CH)])

    pltpu.sync_copy(nodes_hbm.at[pl.ds(base, RPT)], nodes_v)

    pltpu.async_copy(feat_hbm.at[nodes_v], rb0, sem0).wait()
    pltpu.sync_copy(rb0, self_hbm.at[pl.ds(base, RPT)])

    plsc.subcore_barrier()

    pltpu.async_copy(score_s.at[nodes_v], cent_v, sems5).wait()

    for n_hbm, a_hbm in ((n1_hbm, a1_hbm), (n2_hbm, a2_hbm), (n3_hbm, a3_hbm)):
        pltpu.sync_copy(n_hbm.at[pl.ds(base * DEG, RPT * DEG)], neigh_v)

        NQ = (RPT * DEG) // 128
        for q in range(NQ):
            pltpu.async_copy(score_s.at[neigh_v.at[pl.ds(q * 128, 128)]],
                             nsc_v.at[pl.ds(q * 128, 128)], sems5)
        for q in range(NQ):
            pltpu.make_async_copy(scores_hbm.at[pl.ds(0, 128)],
                                  nsc_v.at[pl.ds(q * 128, 128)],
                                  sems5).wait()

        def select_row(j, _):
            i0 = neigh_v[pl.ds(j * DEG, L)]
            i1 = neigh_v[pl.ds(j * DEG + L, L)]
            s0 = nsc_v[pl.ds(j * DEG, L)]
            s1 = nsc_v[pl.ds(j * DEG + L, L)]
            cj = plsc.load_gather(cent_v, [jnp.full((L,), j, jnp.int32)])
            d0 = jnp.abs(s0 - cj)
            d1 = jnp.abs(s1 - cj)
            k0, v0 = plsc.sort_key_val(d0, i0)
            k1, v1 = plsc.sort_key_val(d1, i1)
            rk = lax.rev(k1, (0,))
            rv = lax.rev(v1, (0,))
            sel_v[pl.ds(j * K, K)] = jnp.where(k0 <= rk, v0, rv)
            return 0

        lax.fori_loop(0, RPT, select_row, 0)

        NB = RPT // 8

        def issue(cb, p):
            pltpu.async_copy(feat_hbm.at[sel_v.at[pl.ds(cb * 128, 128)]],
                             rbufs[p], sems[p])

        def drain(p):
            pltpu.make_async_copy(feat_hbm.at[pl.ds(0, 128)],
                                  rbufs[p], sems[p]).wait()

        issue(0, 0)
        issue(1, 1)

        def group_body(g, _):
            for j in range(8):
                cb = g * 8 + j
                issue(jnp.minimum(cb + 2, NB - 1), (j + 2) % 4)
                drain(j % 4)
                buf = rbufs[j % 4]

                def sub_body(sb, _):
                    def row_body(rr, acc):
                        v0 = tuple(buf[sb * 32 + rr, pl.ds(d * L, L)]
                                   for d in range(F // L))
                        v1 = tuple(buf[sb * 32 + K + rr, pl.ds(d * L, L)]
                                   for d in range(F // L))
                        return tuple(a + v for a, v in zip(acc, v0 + v1))

                    acc = lax.fori_loop(
                        0, K, row_body,
                        tuple(jnp.zeros((L,), jnp.float32)
                              for _ in range(16)))
                    for d in range(F // L):
                        agg_v[j * 8 + sb * 2, pl.ds(d * L, L)] = \
                            acc[d] * (1.0 / K)
                        agg_v[j * 8 + sb * 2 + 1, pl.ds(d * L, L)] = \
                            acc[F // L + d] * (1.0 / K)
                    return 0

                lax.fori_loop(0, 4, sub_body, 0)
            pltpu.sync_copy(agg_v, a_hbm.at[pl.ds(base + g * 64, 64)])
            return 0

        lax.fori_loop(0, NB // 8, group_body, 0)
        drain(0)
        drain(1)


_FIN_BLK = 512


def _final_body(sf_ref, a1_ref, a2_ref, a3_ref, clfw_ref, clfb_ref,
                w1a_ref, w1b_ref, w2a_ref, w2b_ref, w3a_ref, w3b_ref,
                wsf_ref, wr1_ref, wr2_ref, wr3_ref,
                comb_ref, cs_ref):
    sf = sf_ref[...]
    cs_ref[...] = (jnp.dot(sf, clfw_ref[...],
                           preferred_element_type=jnp.float32)
                   + clfb_ref[...])

    def rel(a_ref, wa_ref, wb_ref):
        x = (jnp.dot(sf, wa_ref[...], preferred_element_type=jnp.float32)
             + jnp.dot(a_ref[...], wb_ref[...],
                       preferred_element_type=jnp.float32))
        return jnp.maximum(x, 0.0)

    r1 = rel(a1_ref, w1a_ref, w1b_ref)
    r2 = rel(a2_ref, w2a_ref, w2b_ref)
    r3 = rel(a3_ref, w3a_ref, w3b_ref)

    dn = (((0,), (1,)), ((), ()))
    combt = (lax.dot_general(wsf_ref[...], sf, dn,
                             preferred_element_type=jnp.float32)
             + lax.dot_general(wr1_ref[...], r1, dn,
                               preferred_element_type=jnp.float32)
             + lax.dot_general(wr2_ref[...], r2, dn,
                               preferred_element_type=jnp.float32)
             + lax.dot_general(wr3_ref[...], r3, dn,
                               preferred_element_type=jnp.float32))
    comb_ref[...] = jnp.maximum(combt, 0.0)


_final = pl.pallas_call(
    _final_body,
    grid=(B // _FIN_BLK,),
    in_specs=[
        pl.BlockSpec((_FIN_BLK, F), lambda i: (i, 0)),
        pl.BlockSpec((_FIN_BLK, F), lambda i: (i, 0)),
        pl.BlockSpec((_FIN_BLK, F), lambda i: (i, 0)),
        pl.BlockSpec((_FIN_BLK, F), lambda i: (i, 0)),
        pl.BlockSpec((F, 2), lambda i: (0, 0)),
        pl.BlockSpec((1, 2), lambda i: (0, 0)),
        pl.BlockSpec((F, E), lambda i: (0, 0)),
        pl.BlockSpec((F, E), lambda i: (0, 0)),
        pl.BlockSpec((F, E), lambda i: (0, 0)),
        pl.BlockSpec((F, E), lambda i: (0, 0)),
        pl.BlockSpec((F, E), lambda i: (0, 0)),
        pl.BlockSpec((F, E), lambda i: (0, 0)),
        pl.BlockSpec((F, E), lambda i: (0, 0)),
        pl.BlockSpec((E, E), lambda i: (0, 0)),
        pl.BlockSpec((E, E), lambda i: (0, 0)),
        pl.BlockSpec((E, E), lambda i: (0, 0)),
    ],
    out_specs=[
        pl.BlockSpec((E, _FIN_BLK), lambda i: (0, i)),
        pl.BlockSpec((_FIN_BLK, 2), lambda i: (i, 0)),
    ],
    out_shape=[
        jax.ShapeDtypeStruct((E, B), jnp.float32),
        jax.ShapeDtypeStruct((B, 2), jnp.float32),
    ],
)


def kernel(nodes, labels, neigh1, neigh2, neigh3, train_pos, feat_table,
           clf_w, clf_b, w1, w2, w3, weight):
    del labels, train_pos
    nodes = nodes.astype(jnp.int32)
    neigh1 = neigh1.astype(jnp.int32).reshape(B * DEG)
    neigh2 = neigh2.astype(jnp.int32).reshape(B * DEG)
    neigh3 = neigh3.astype(jnp.int32).reshape(B * DEG)

    scores = _score_scan(feat_table, clf_w[:, 0:1])
    self_feats, a1, a2, a3 = _sc_select_agg(
        scores, nodes, neigh1, neigh2, neigh3, feat_table)
    combined, center_scores = _final(
        self_feats, a1, a2, a3, clf_w, clf_b.reshape(1, 2),
        w1[:F], w1[F:], w2[:F], w2[F:], w3[:F], w3[F:],
        weight[:F], weight[F:F + E], weight[F + E:F + 2 * E],
        weight[F + 2 * E:])
    return combined, center_scores


# speedup vs baseline: 1.0743x; 1.0743x over previous
import functools

import jax
import jax.numpy as jnp
from jax import lax
from jax.experimental import pallas as pl
from jax.experimental.pallas import tpu as pltpu
from jax.experimental.pallas import tpu_sc as plsc

N_NODES = 100000
F = 128
E = 64
B = 4096
DEG = 32
K = 16
L = 16
NC, NS = 2, 16
NW = NC * NS
RPT = B // NW

_SCORE_BLK = 8192
_N_PAD = 106496


def _score_body(ft_ref, w_ref, out_ref):
    res = lax.dot_general(w_ref[...], ft_ref[...], (((0,), (1,)), ((), ())),
                          preferred_element_type=jnp.float32)
    out_ref[...] = res[0]


_score_scan = pl.pallas_call(
    _score_body,
    grid=(_N_PAD // _SCORE_BLK,),
    in_specs=[
        pl.BlockSpec((_SCORE_BLK, F), lambda i: (i, 0)),
        pl.BlockSpec((F, 1), lambda i: (0, 0)),
    ],
    out_specs=pl.BlockSpec((_SCORE_BLK,), lambda i: (i,)),
    out_shape=jax.ShapeDtypeStruct((_N_PAD,), jnp.float32),
)

_sc_mesh = plsc.VectorSubcoreMesh(core_axis_name="c", subcore_axis_name="s")


@functools.partial(
    pl.kernel,
    out_type=[
        jax.ShapeDtypeStruct((B, F), jnp.float32),
        jax.ShapeDtypeStruct((B, F), jnp.float32),
        jax.ShapeDtypeStruct((B, F), jnp.float32),
        jax.ShapeDtypeStruct((B, F), jnp.float32),
    ],
    mesh=_sc_mesh,
    compiler_params=pltpu.CompilerParams(needs_layout_passes=False),
    scratch_types=[
        pltpu.VMEM((RPT,), jnp.int32),
        pltpu.VMEM((RPT,), jnp.float32),
        pltpu.VMEM((RPT * DEG,), jnp.int32),
        pltpu.VMEM((RPT * DEG,), jnp.float32),
        pltpu.VMEM((RPT * K,), jnp.int32),
        pltpu.VMEM((128, F), jnp.float32),
        pltpu.VMEM((128, F), jnp.float32),
        pltpu.VMEM((128, F), jnp.float32),
        pltpu.VMEM((128, F), jnp.float32),
        pltpu.VMEM((64, F), jnp.float32),
        pltpu.VMEM_SHARED((_N_PAD,), jnp.float32),
        pltpu.SemaphoreType.DMA,
        pltpu.SemaphoreType.DMA,
        pltpu.SemaphoreType.DMA,
        pltpu.SemaphoreType.DMA,
        pltpu.SemaphoreType.DMA,
    ],
)
def _sc_select_agg(scores_hbm, nodes_hbm, n1_hbm, n2_hbm, n3_hbm, feat_hbm,
                   self_hbm, a1_hbm, a2_hbm, a3_hbm,
                   nodes_v, cent_v, neigh_v, nsc_v, sel_v,
                   rb0, rb1, rb2, rb3, agg_v, score_s,
                   sem0, sem1, sem2, sem3, sems5):
    sid = lax.axis_index("s")
    wid = sid * NC + lax.axis_index("c")
    base = wid * RPT

    rbufs = (rb0, rb1, rb2, rb3)
    sems = (sem0, sem1, sem2, sem3)

    _CH = _N_PAD // NS
    pltpu.sync_copy(scores_hbm.at[pl.ds(sid * _CH, _CH)],
                    score_s.at[pl.ds(sid * _CH, _CH)])

    pltpu.sync_copy(nodes_hbm.at[pl.ds(base, RPT)], nodes_v)

    pltpu.async_copy(feat_hbm.at[nodes_v], rb0, sem0).wait()
    pltpu.sync_copy(rb0, self_hbm.at[pl.ds(base, RPT)])

    plsc.subcore_barrier()

    pltpu.async_copy(score_s.at[nodes_v], cent_v, sems5).wait()

    for n_hbm, a_hbm in ((n1_hbm, a1_hbm), (n2_hbm, a2_hbm), (n3_hbm, a3_hbm)):
        pltpu.sync_copy(n_hbm.at[pl.ds(base * DEG, RPT * DEG)], neigh_v)

        NQ = (RPT * DEG) // 128
        for q in range(NQ):
            pltpu.async_copy(score_s.at[neigh_v.at[pl.ds(q * 128, 128)]],
                             nsc_v.at[pl.ds(q * 128, 128)], sems5)
        for q in range(NQ):
            pltpu.make_async_copy(scores_hbm.at[pl.ds(0, 128)],
                                  nsc_v.at[pl.ds(q * 128, 128)],
                                  sems5).wait()

        def select_row(j, _):
            i0 = neigh_v[pl.ds(j * DEG, L)]
            i1 = neigh_v[pl.ds(j * DEG + L, L)]
            s0 = nsc_v[pl.ds(j * DEG, L)]
            s1 = nsc_v[pl.ds(j * DEG + L, L)]
            cj = plsc.load_gather(cent_v, [jnp.full((L,), j, jnp.int32)])
            d0 = jnp.abs(s0 - cj)
            d1 = jnp.abs(s1 - cj)
            k0, v0 = plsc.sort_key_val(d0, i0)
            k1, v1 = plsc.sort_key_val(d1, i1)
            rk = lax.rev(k1, (0,))
            rv = lax.rev(v1, (0,))
            sel_v[pl.ds(j * K, K)] = jnp.where(k0 <= rk, v0, rv)
            return 0

        lax.fori_loop(0, RPT, select_row, 0)

        NB = RPT // 8

        def issue(cb, p):
            pltpu.async_copy(feat_hbm.at[sel_v.at[pl.ds(cb * 128, 128)]],
                             rbufs[p], sems[p])

        def drain(p):
            pltpu.make_async_copy(feat_hbm.at[pl.ds(0, 128)],
                                  rbufs[p], sems[p]).wait()

        issue(0, 0)
        issue(1, 1)

        def group_body(g, _):
            for j in range(8):
                cb = g * 8 + j
                issue(jnp.minimum(cb + 2, NB - 1), (j + 2) % 4)
                drain(j % 4)
                buf = rbufs[j % 4]

                def sub_body(sb, _):
                    def row_body(rr, acc):
                        v0 = tuple(buf[sb * 32 + rr, pl.ds(d * L, L)]
                                   for d in range(F // L))
                        v1 = tuple(buf[sb * 32 + K + rr, pl.ds(d * L, L)]
                                   for d in range(F // L))
                        return tuple(a + v for a, v in zip(acc, v0 + v1))

                    acc = lax.fori_loop(
                        0, K, row_body,
                        tuple(jnp.zeros((L,), jnp.float32)
                              for _ in range(16)))
                    for d in range(F // L):
                        agg_v[j * 8 + sb * 2, pl.ds(d * L, L)] = \
                            acc[d] * (1.0 / K)
                        agg_v[j * 8 + sb * 2 + 1, pl.ds(d * L, L)] = \
                            acc[F // L + d] * (1.0 / K)
                    return 0

                lax.fori_loop(0, 4, sub_body, 0)
            pltpu.sync_copy(agg_v, a_hbm.at[pl.ds(base + g * 64, 64)])
            return 0

        lax.fori_loop(0, NB // 8, group_body, 0)
        drain(0)
        drain(1)


_FIN_BLK = 1024


def _final_body(sf_ref, a1_ref, a2_ref, a3_ref, clfw_ref, clfb_ref,
                w1a_ref, w1b_ref, w2a_ref, w2b_ref, w3a_ref, w3b_ref,
                wsf_ref, wr1_ref, wr2_ref, wr3_ref,
                comb_ref, cs_ref):
    sf = sf_ref[...]
    cs_ref[...] = (jnp.dot(sf, clfw_ref[...],
                           preferred_element_type=jnp.float32)
                   + clfb_ref[...])

    def rel(a_ref, wa_ref, wb_ref):
        x = (jnp.dot(sf, wa_ref[...], preferred_element_type=jnp.float32)
             + jnp.dot(a_ref[...], wb_ref[...],
                       preferred_element_type=jnp.float32))
        return jnp.maximum(x, 0.0)

    r1 = rel(a1_ref, w1a_ref, w1b_ref)
    r2 = rel(a2_ref, w2a_ref, w2b_ref)
    r3 = rel(a3_ref, w3a_ref, w3b_ref)

    dn = (((0,), (1,)), ((), ()))
    combt = (lax.dot_general(wsf_ref[...], sf, dn,
                             preferred_element_type=jnp.float32)
             + lax.dot_general(wr1_ref[...], r1, dn,
                               preferred_element_type=jnp.float32)
             + lax.dot_general(wr2_ref[...], r2, dn,
                               preferred_element_type=jnp.float32)
             + lax.dot_general(wr3_ref[...], r3, dn,
                               preferred_element_type=jnp.float32))
    comb_ref[...] = jnp.maximum(combt, 0.0)


_final = pl.pallas_call(
    _final_body,
    grid=(B // _FIN_BLK,),
    in_specs=[
        pl.BlockSpec((_FIN_BLK, F), lambda i: (i, 0)),
        pl.BlockSpec((_FIN_BLK, F), lambda i: (i, 0)),
        pl.BlockSpec((_FIN_BLK, F), lambda i: (i, 0)),
        pl.BlockSpec((_FIN_BLK, F), lambda i: (i, 0)),
        pl.BlockSpec((F, 2), lambda i: (0, 0)),
        pl.BlockSpec((1, 2), lambda i: (0, 0)),
        pl.BlockSpec((F, E), lambda i: (0, 0)),
        pl.BlockSpec((F, E), lambda i: (0, 0)),
        pl.BlockSpec((F, E), lambda i: (0, 0)),
        pl.BlockSpec((F, E), lambda i: (0, 0)),
        pl.BlockSpec((F, E), lambda i: (0, 0)),
        pl.BlockSpec((F, E), lambda i: (0, 0)),
        pl.BlockSpec((F, E), lambda i: (0, 0)),
        pl.BlockSpec((E, E), lambda i: (0, 0)),
        pl.BlockSpec((E, E), lambda i: (0, 0)),
        pl.BlockSpec((E, E), lambda i: (0, 0)),
    ],
    out_specs=[
        pl.BlockSpec((E, _FIN_BLK), lambda i: (0, i)),
        pl.BlockSpec((_FIN_BLK, 2), lambda i: (i, 0)),
    ],
    out_shape=[
        jax.ShapeDtypeStruct((E, B), jnp.float32),
        jax.ShapeDtypeStruct((B, 2), jnp.float32),
    ],
)


def kernel(nodes, labels, neigh1, neigh2, neigh3, train_pos, feat_table,
           clf_w, clf_b, w1, w2, w3, weight):
    del labels, train_pos
    nodes = nodes.astype(jnp.int32)
    neigh1 = neigh1.astype(jnp.int32).reshape(B * DEG)
    neigh2 = neigh2.astype(jnp.int32).reshape(B * DEG)
    neigh3 = neigh3.astype(jnp.int32).reshape(B * DEG)

    scores = _score_scan(feat_table, clf_w[:, 0:1])
    self_feats, a1, a2, a3 = _sc_select_agg(
        scores, nodes, neigh1, neigh2, neigh3, feat_table)
    combined, center_scores = _final(
        self_feats, a1, a2, a3, clf_w, clf_b.reshape(1, 2),
        w1[:F], w1[F:], w2[:F], w2[F:], w3[:F], w3[F:],
        weight[:F], weight[F:F + E], weight[F + E:F + 2 * E],
        weight[F + 2 * E:])
    return combined, center_scores
